# baseline (speedup 1.0000x reference)
import functools

import jax
import jax.numpy as jnp
from jax import lax
from jax.experimental import pallas as pl
from jax.experimental.pallas import tpu as pltpu
from jax.experimental.pallas import tpu_sc as plsc


_BN = 256
_NC, _NS = 2, 16
_NW = _NC * _NS
_LANES = 16


def _scores_body(expert_map_ref, x_ref, w_ref, out_ref):
    s = lax.dot_general(
        w_ref[0], x_ref[...],
        dimension_numbers=(((0,), (1,)), ((), ())),
        preferred_element_type=jnp.float32,
    )
    m = jnp.max(s, axis=0, keepdims=True)
    p = jnp.exp(s - m)
    out_ref[...] = p / jnp.sum(p, axis=0, keepdims=True)


def _probs_transposed(x, w, grouped_gemm_batch_sizes):
    n, h = x.shape
    e, _, l = w.shape
    num_blocks = n // _BN
    cum = jnp.cumsum(grouped_gemm_batch_sizes)
    starts = jnp.arange(num_blocks, dtype=jnp.int32) * _BN
    expert_map = jnp.searchsorted(cum, starts, side="right").astype(jnp.int32)

    grid_spec = pltpu.PrefetchScalarGridSpec(
        num_scalar_prefetch=1,
        grid=(num_blocks,),
        in_specs=[
            pl.BlockSpec((_BN, h), lambda i, m: (i, 0)),
            pl.BlockSpec((1, h, l), lambda i, m: (m[i], 0, 0)),
        ],
        out_specs=pl.BlockSpec((l, _BN), lambda i, m: (0, i)),
    )
    return pl.pallas_call(
        _scores_body,
        grid_spec=grid_spec,
        out_shape=jax.ShapeDtypeStruct((l, n), jnp.float32),
    )(expert_map, x, w)


def _top2_body(probs_hbm, wout_hbm, iout_hbm, pv, wbuf, ibuf):
    l, n = probs_hbm.shape
    chunk = n // _NW
    groups = chunk // _LANES
    wid = lax.axis_index("s") * _NC + lax.axis_index("c")
    base = wid * chunk
    pltpu.sync_copy(probs_hbm.at[:, pl.ds(base, chunk)], pv)

    def per_group(g, carry):
        off = g * _LANES
        m1 = pv[0, pl.ds(off, _LANES)]
        i1 = jnp.zeros((_LANES,), jnp.int32)
        m2 = jnp.full((_LANES,), -1.0, jnp.float32)
        i2 = jnp.zeros((_LANES,), jnp.int32)
        for c in range(1, l):
            v = pv[c, pl.ds(off, _LANES)]
            cv = jnp.full((_LANES,), c, jnp.int32)
            gt1 = v > m1
            gt2 = v > m2
            m2 = jnp.where(gt1, m1, jnp.where(gt2, v, m2))
            i2 = jnp.where(gt1, i1, jnp.where(gt2, cv, i2))
            m1 = jnp.where(gt1, v, m1)
            i1 = jnp.where(gt1, cv, i1)
        wbuf[0, pl.ds(off, _LANES)] = m1
        wbuf[1, pl.ds(off, _LANES)] = m2
        ibuf[0, pl.ds(off, _LANES)] = i1
        ibuf[1, pl.ds(off, _LANES)] = i2
        return carry

    lax.fori_loop(0, groups, per_group, 0)
    pltpu.sync_copy(wbuf, wout_hbm.at[:, pl.ds(base, chunk)])
    pltpu.sync_copy(ibuf, iout_hbm.at[:, pl.ds(base, chunk)])


def _top2_route(probs_t):
    l, n = probs_t.shape
    chunk = n // _NW
    mesh = plsc.VectorSubcoreMesh(
        core_axis_name="c", subcore_axis_name="s",
        num_cores=_NC, num_subcores=_NS,
    )
    w2, i2 = pl.kernel(
        _top2_body,
        out_type=[
            jax.ShapeDtypeStruct((2, n), jnp.float32),
            jax.ShapeDtypeStruct((2, n), jnp.int32),
        ],
        mesh=mesh,
        scratch_types=[
            pltpu.VMEM((l, chunk), jnp.float32),
            pltpu.VMEM((2, chunk), jnp.float32),
            pltpu.VMEM((2, chunk), jnp.int32),
        ],
    )(probs_t)
    return w2.T, i2.T


@jax.jit
def kernel(x, w1, grouped_gemm_batch_sizes):
    n, h = x.shape
    e = grouped_gemm_batch_sizes.shape[0]
    l = w1.shape[0] // e
    w = w1.reshape(e, h, l)
    probs_t = _probs_transposed(x, w, grouped_gemm_batch_sizes)
    return _top2_route(probs_t)


# speedup vs baseline: 1.1568x; 1.1568x over previous
import functools

import jax
import jax.numpy as jnp
from jax import lax
from jax.experimental import pallas as pl
from jax.experimental.pallas import tpu as pltpu
from jax.experimental.pallas import tpu_sc as plsc


_BN = 512
_NC, _NS = 2, 16
_NW = _NC * _NS
_LANES = 16


def _scores_body(expert_map_ref, x_ref, w_ref, out_ref):
    s = lax.dot_general(
        w_ref[0], x_ref[...],
        dimension_numbers=(((0,), (1,)), ((), ())),
        preferred_element_type=jnp.float32,
    )
    m = jnp.max(s, axis=0, keepdims=True)
    p = jnp.exp(s - m)
    out_ref[...] = p / jnp.sum(p, axis=0, keepdims=True)


def _probs_transposed(x, w, grouped_gemm_batch_sizes):
    n, h = x.shape
    e, _, l = w.shape
    num_blocks = n // _BN
    cum = jnp.cumsum(grouped_gemm_batch_sizes)
    starts = jnp.arange(num_blocks, dtype=jnp.int32) * _BN
    expert_map = jnp.searchsorted(cum, starts, side="right").astype(jnp.int32)

    grid_spec = pltpu.PrefetchScalarGridSpec(
        num_scalar_prefetch=1,
        grid=(num_blocks,),
        in_specs=[
            pl.BlockSpec((_BN, h), lambda i, m: (i, 0)),
            pl.BlockSpec((1, h, l), lambda i, m: (m[i], 0, 0)),
        ],
        out_specs=pl.BlockSpec((l, _BN), lambda i, m: (0, i)),
    )
    return pl.pallas_call(
        _scores_body,
        grid_spec=grid_spec,
        out_shape=jax.ShapeDtypeStruct((l, n), jnp.float32),
    )(expert_map, x, w)


def _top2_body(probs_hbm, wout_hbm, iout_hbm, pv, wbuf, ibuf):
    l, n = probs_hbm.shape
    chunk = n // _NW
    groups = chunk // _LANES
    wid = lax.axis_index("s") * _NC + lax.axis_index("c")
    base = wid * chunk
    pltpu.sync_copy(probs_hbm.at[:, pl.ds(base, chunk)], pv)

    def per_group(g, carry):
        off = g * _LANES
        m1 = pv[0, pl.ds(off, _LANES)]
        i1 = jnp.zeros((_LANES,), jnp.int32)
        m2 = jnp.full((_LANES,), -1.0, jnp.float32)
        i2 = jnp.zeros((_LANES,), jnp.int32)
        for c in range(1, l):
            v = pv[c, pl.ds(off, _LANES)]
            cv = jnp.full((_LANES,), c, jnp.int32)
            gt1 = v > m1
            gt2 = v > m2
            m2 = jnp.where(gt1, m1, jnp.where(gt2, v, m2))
            i2 = jnp.where(gt1, i1, jnp.where(gt2, cv, i2))
            m1 = jnp.where(gt1, v, m1)
            i1 = jnp.where(gt1, cv, i1)
        wbuf[0, pl.ds(off, _LANES)] = m1
        wbuf[1, pl.ds(off, _LANES)] = m2
        ibuf[0, pl.ds(off, _LANES)] = i1
        ibuf[1, pl.ds(off, _LANES)] = i2
        return carry

    lax.fori_loop(0, groups, per_group, 0)
    pltpu.sync_copy(wbuf, wout_hbm.at[:, pl.ds(base, chunk)])
    pltpu.sync_copy(ibuf, iout_hbm.at[:, pl.ds(base, chunk)])


def _top2_route(probs_t):
    l, n = probs_t.shape
    chunk = n // _NW
    mesh = plsc.VectorSubcoreMesh(
        core_axis_name="c", subcore_axis_name="s",
        num_cores=_NC, num_subcores=_NS,
    )
    w2, i2 = pl.kernel(
        _top2_body,
        out_type=[
            jax.ShapeDtypeStruct((2, n), jnp.float32),
            jax.ShapeDtypeStruct((2, n), jnp.int32),
        ],
        mesh=mesh,
        scratch_types=[
            pltpu.VMEM((l, chunk), jnp.float32),
            pltpu.VMEM((2, chunk), jnp.float32),
            pltpu.VMEM((2, chunk), jnp.int32),
        ],
    )(probs_t)
    return w2.T, i2.T


@jax.jit
def kernel(x, w1, grouped_gemm_batch_sizes):
    n, h = x.shape
    e = grouped_gemm_batch_sizes.shape[0]
    l = w1.shape[0] // e
    w = w1.reshape(e, h, l)
    probs_t = _probs_transposed(x, w, grouped_gemm_batch_sizes)
    return _top2_route(probs_t)


# speedup vs baseline: 1.2397x; 1.0717x over previous
import functools

import jax
import jax.numpy as jnp
from jax import lax
from jax.experimental import pallas as pl
from jax.experimental.pallas import tpu as pltpu
from jax.experimental.pallas import tpu_sc as plsc


_BN = 1024
_NC, _NS = 2, 16
_NW = _NC * _NS
_LANES = 16


def _scores_body(expert_map_ref, x_ref, w_ref, out_ref):
    s = lax.dot_general(
        w_ref[0], x_ref[...],
        dimension_numbers=(((0,), (1,)), ((), ())),
        preferred_element_type=jnp.float32,
    )
    m = jnp.max(s, axis=0, keepdims=True)
    p = jnp.exp(s - m)
    out_ref[...] = p / jnp.sum(p, axis=0, keepdims=True)


def _probs_transposed(x, w, grouped_gemm_batch_sizes):
    n, h = x.shape
    e, _, l = w.shape
    num_blocks = n // _BN
    cum = jnp.cumsum(grouped_gemm_batch_sizes)
    starts = jnp.arange(num_blocks, dtype=jnp.int32) * _BN
    expert_map = jnp.searchsorted(cum, starts, side="right").astype(jnp.int32)

    grid_spec = pltpu.PrefetchScalarGridSpec(
        num_scalar_prefetch=1,
        grid=(num_blocks,),
        in_specs=[
            pl.BlockSpec((_BN, h), lambda i, m: (i, 0)),
            pl.BlockSpec((1, h, l), lambda i, m: (m[i], 0, 0)),
        ],
        out_specs=pl.BlockSpec((l, _BN), lambda i, m: (0, i)),
    )
    return pl.pallas_call(
        _scores_body,
        grid_spec=grid_spec,
        out_shape=jax.ShapeDtypeStruct((l, n), jnp.float32),
    )(expert_map, x, w)


def _top2_body(probs_hbm, wout_hbm, iout_hbm, pv, wbuf, ibuf):
    l, n = probs_hbm.shape
    chunk = n // _NW
    groups = chunk // _LANES
    wid = lax.axis_index("s") * _NC + lax.axis_index("c")
    base = wid * chunk
    pltpu.sync_copy(probs_hbm.at[:, pl.ds(base, chunk)], pv)

    def per_group(g, carry):
        off = g * _LANES
        m1 = pv[0, pl.ds(off, _LANES)]
        i1 = jnp.zeros((_LANES,), jnp.int32)
        m2 = jnp.full((_LANES,), -1.0, jnp.float32)
        i2 = jnp.zeros((_LANES,), jnp.int32)
        for c in range(1, l):
            v = pv[c, pl.ds(off, _LANES)]
            cv = jnp.full((_LANES,), c, jnp.int32)
            gt1 = v > m1
            gt2 = v > m2
            m2 = jnp.where(gt1, m1, jnp.where(gt2, v, m2))
            i2 = jnp.where(gt1, i1, jnp.where(gt2, cv, i2))
            m1 = jnp.where(gt1, v, m1)
            i1 = jnp.where(gt1, cv, i1)
        wbuf[0, pl.ds(off, _LANES)] = m1
        wbuf[1, pl.ds(off, _LANES)] = m2
        ibuf[0, pl.ds(off, _LANES)] = i1
        ibuf[1, pl.ds(off, _LANES)] = i2
        return carry

    lax.fori_loop(0, groups, per_group, 0)
    pltpu.sync_copy(wbuf, wout_hbm.at[:, pl.ds(base, chunk)])
    pltpu.sync_copy(ibuf, iout_hbm.at[:, pl.ds(base, chunk)])


def _top2_route(probs_t):
    l, n = probs_t.shape
    chunk = n // _NW
    mesh = plsc.VectorSubcoreMesh(
        core_axis_name="c", subcore_axis_name="s",
        num_cores=_NC, num_subcores=_NS,
    )
    w2, i2 = pl.kernel(
        _top2_body,
        out_type=[
            jax.ShapeDtypeStruct((2, n), jnp.float32),
            jax.ShapeDtypeStruct((2, n), jnp.int32),
        ],
        mesh=mesh,
        scratch_types=[
            pltpu.VMEM((l, chunk), jnp.float32),
            pltpu.VMEM((2, chunk), jnp.float32),
            pltpu.VMEM((2, chunk), jnp.int32),
        ],
    )(probs_t)
    return w2.T, i2.T


@jax.jit
def kernel(x, w1, grouped_gemm_batch_sizes):
    n, h = x.shape
    e = grouped_gemm_batch_sizes.shape[0]
    l = w1.shape[0] // e
    w = w1.reshape(e, h, l)
    probs_t = _probs_transposed(x, w, grouped_gemm_batch_sizes)
    return _top2_route(probs_t)
